# baseline (speedup 1.0000x reference)
import jax
import jax.numpy as jnp
from jax import lax
from jax.experimental import pallas as pl
from jax.experimental.pallas import tpu as pltpu
from jax.experimental.pallas import tpu_sc as plsc

HIDDEN = 1024
EXPERTS = 64
TOPK = 8
TOKENS = 32768
BLOCK = 512

NUM_CORES = 2
NUM_SUBCORES = 16
LANES = 16
NW = NUM_CORES * NUM_SUBCORES
ROWS_PER_W = TOKENS // NW
CHUNK = 256
GROUPS = CHUNK // LANES
NCHUNK = ROWS_PER_W // CHUNK


def _logits_kernel(x_ref, w_ref, b_ref, logits_ref):
    x = x_ref[...]
    w = w_ref[...]
    acc = jax.lax.dot_general(
        x, w, (((1,), (1,)), ((), ())), preferred_element_type=jnp.float32
    )
    logits_ref[...] = acc + b_ref[...]


def _compute_logits(x, W, b):
    b2 = b.reshape(1, EXPERTS)
    return pl.pallas_call(
        _logits_kernel,
        grid=(TOKENS // BLOCK,),
        in_specs=[
            pl.BlockSpec((BLOCK, HIDDEN), lambda i: (i, 0)),
            pl.BlockSpec((EXPERTS, HIDDEN), lambda i: (0, 0)),
            pl.BlockSpec((1, EXPERTS), lambda i: (0, 0)),
        ],
        out_specs=pl.BlockSpec((BLOCK, EXPERTS), lambda i: (i, 0)),
        out_shape=jax.ShapeDtypeStruct((TOKENS, EXPERTS), jnp.float32),
        compiler_params=pltpu.CompilerParams(
            dimension_semantics=("arbitrary",),
        ),
    )(x, W, b2)


def _route_body(logits_hbm, sparse_hbm, idx_hbm, in_v, out_v, idx_v):
    c = lax.axis_index("c")
    s = lax.axis_index("s")
    wid = s * NUM_CORES + c
    base = wid * ROWS_PER_W
    lane_iota = lax.iota(jnp.int32, LANES)
    zero16 = jnp.zeros((LANES,), jnp.float32)
    neg16 = jnp.full((LANES,), -jnp.inf, jnp.float32)
    izero16 = jnp.zeros((LANES,), jnp.int32)

    def chunk_body(ci, carry):
        row0 = base + ci * CHUNK
        pltpu.sync_copy(logits_hbm.at[pl.ds(row0 * EXPERTS, CHUNK * EXPERTS)], in_v)

        def group_body(g, carry2):
            rows = g * LANES + lane_iota

            def zero_body(z, _):
                out_v[pl.ds(g * LANES * EXPERTS + z * LANES, LANES)] = zero16
                return 0

            lax.fori_loop(0, LANES * EXPERTS // LANES, zero_body, 0)

            def exp_body(e, tk):
                vs = list(tk[:TOPK])
                ix = list(tk[TOPK:])
                t = plsc.load_gather(in_v, [rows * EXPERTS + e])
                ti = izero16 + e
                for j in range(TOPK):
                    cgt = t > vs[j]
                    nv = jnp.maximum(vs[j], t)
                    nt = jnp.minimum(vs[j], t)
                    ni = jnp.where(cgt, ti, ix[j])
                    nti = jnp.where(cgt, ix[j], ti)
                    vs[j], t, ix[j], ti = nv, nt, ni, nti
                return tuple(vs) + tuple(ix)

            init = tuple([neg16] * TOPK) + tuple([izero16] * TOPK)
            tk = lax.fori_loop(0, EXPERTS, exp_body, init)
            vs = tk[:TOPK]
            ix = tk[TOPK:]

            m0 = vs[0]
            es = [jnp.exp(v - m0) for v in vs]
            tot = es[0]
            for j in range(1, TOPK):
                tot = tot + es[j]
            inv = 1.0 / tot
            for j in range(TOPK):
                pj = es[j] * inv
                plsc.store_scatter(out_v, [rows * EXPERTS + ix[j]], pj)
                plsc.store_scatter(idx_v, [rows * TOPK + j], ix[j])
            return carry2

        lax.fori_loop(0, GROUPS, group_body, 0)
        pltpu.sync_copy(out_v, sparse_hbm.at[pl.ds(row0 * EXPERTS, CHUNK * EXPERTS)])
        pltpu.sync_copy(idx_v, idx_hbm.at[pl.ds(row0 * TOPK, CHUNK * TOPK)])
        return carry

    lax.fori_loop(0, NCHUNK, chunk_body, 0)


def _route(logits_flat):
    mesh = plsc.VectorSubcoreMesh(
        core_axis_name="c",
        subcore_axis_name="s",
        num_cores=NUM_CORES,
        num_subcores=NUM_SUBCORES,
    )
    fn = pl.kernel(
        _route_body,
        out_type=[
            jax.ShapeDtypeStruct((TOKENS * EXPERTS,), jnp.float32),
            jax.ShapeDtypeStruct((TOKENS * TOPK,), jnp.int32),
        ],
        mesh=mesh,
        scratch_types=[
            pltpu.VMEM((CHUNK * EXPERTS,), jnp.float32),
            pltpu.VMEM((CHUNK * EXPERTS,), jnp.float32),
            pltpu.VMEM((CHUNK * TOPK,), jnp.int32),
        ],
        compiler_params=pltpu.CompilerParams(needs_layout_passes=False),
    )
    return fn(logits_flat)


@jax.jit
def kernel(x, W, b):
    logits = _compute_logits(x, W, b)
    sparse_flat, idx_flat = _route(logits.reshape(-1))
    return (
        sparse_flat.reshape(TOKENS, EXPERTS),
        idx_flat.reshape(TOKENS, TOPK),
        logits,
    )


# speedup vs baseline: 1.0311x; 1.0311x over previous
import jax
import jax.numpy as jnp
from jax import lax
from jax.experimental import pallas as pl
from jax.experimental.pallas import tpu as pltpu
from jax.experimental.pallas import tpu_sc as plsc

HIDDEN = 1024
EXPERTS = 64
TOPK = 8
TOKENS = 32768
BLOCK = 512

NUM_CORES = 2
NUM_SUBCORES = 16
LANES = 16
NW = NUM_CORES * NUM_SUBCORES
ROWS_PER_W = TOKENS // NW
CHUNK = 256
GROUPS = CHUNK // LANES
NCHUNK = ROWS_PER_W // CHUNK


def _logits_kernel(x_ref, w_ref, b_ref, logits_ref):
    x = x_ref[...]
    w = w_ref[...]
    acc = jax.lax.dot_general(
        x, w, (((1,), (1,)), ((), ())), preferred_element_type=jnp.float32
    )
    logits_ref[...] = acc + b_ref[...]


def _compute_logits(x, W, b):
    b2 = b.reshape(1, EXPERTS)
    return pl.pallas_call(
        _logits_kernel,
        grid=(TOKENS // BLOCK,),
        in_specs=[
            pl.BlockSpec((BLOCK, HIDDEN), lambda i: (i, 0)),
            pl.BlockSpec((EXPERTS, HIDDEN), lambda i: (0, 0)),
            pl.BlockSpec((1, EXPERTS), lambda i: (0, 0)),
        ],
        out_specs=pl.BlockSpec((BLOCK, EXPERTS), lambda i: (i, 0)),
        out_shape=jax.ShapeDtypeStruct((TOKENS, EXPERTS), jnp.float32),
        compiler_params=pltpu.CompilerParams(
            dimension_semantics=("arbitrary",),
        ),
    )(x, W, b2)


def _route_body(logits_hbm, sparse_hbm, idx_hbm, in_v, out_v, idx_v):
    c = lax.axis_index("c")
    s = lax.axis_index("s")
    wid = s * NUM_CORES + c
    base = wid * ROWS_PER_W
    lane_iota = lax.iota(jnp.int32, LANES)
    zero16 = jnp.zeros((LANES,), jnp.float32)
    neg16 = jnp.full((LANES,), -jnp.inf, jnp.float32)
    izero16 = jnp.zeros((LANES,), jnp.int32)

    def chunk_body(ci, carry):
        row0 = base + ci * CHUNK
        pltpu.sync_copy(logits_hbm.at[pl.ds(row0 * EXPERTS, CHUNK * EXPERTS)], in_v)

        def group_body(g, carry2):
            rows = g * LANES + lane_iota
            rbase = rows * EXPERTS

            def zero_body(z, _):
                out_v[pl.ds(g * LANES * EXPERTS + z * LANES * 4, LANES)] = zero16
                out_v[pl.ds(g * LANES * EXPERTS + z * LANES * 4 + LANES, LANES)] = zero16
                out_v[pl.ds(g * LANES * EXPERTS + z * LANES * 4 + 2 * LANES, LANES)] = zero16
                out_v[pl.ds(g * LANES * EXPERTS + z * LANES * 4 + 3 * LANES, LANES)] = zero16
                return 0

            lax.fori_loop(0, LANES * EXPERTS // (LANES * 4), zero_body, 0)

            UNROLL = 8

            def exp_body(eo, tk):
                vs = list(tk[:TOPK])
                ix = list(tk[TOPK:])
                ebase = izero16 + eo * UNROLL
                for k in range(UNROLL):
                    t = plsc.load_gather(in_v, [rbase + (eo * UNROLL + k)])
                    ti = ebase + k
                    for j in range(TOPK):
                        cgt = t > vs[j]
                        nv = jnp.maximum(vs[j], t)
                        nt = jnp.minimum(vs[j], t)
                        ni = jnp.where(cgt, ti, ix[j])
                        nti = jnp.where(cgt, ix[j], ti)
                        vs[j], t, ix[j], ti = nv, nt, ni, nti
                return tuple(vs) + tuple(ix)

            init = tuple([neg16] * TOPK) + tuple([izero16] * TOPK)
            tk = lax.fori_loop(0, EXPERTS // UNROLL, exp_body, init)
            vs = tk[:TOPK]
            ix = tk[TOPK:]

            m0 = vs[0]
            es = [jnp.exp(v - m0) for v in vs]
            tot = es[0]
            for j in range(1, TOPK):
                tot = tot + es[j]
            inv = 1.0 / tot
            for j in range(TOPK):
                pj = es[j] * inv
                plsc.store_scatter(out_v, [rbase + ix[j]], pj)
                plsc.store_scatter(idx_v, [rows * TOPK + j], ix[j])
            return carry2

        lax.fori_loop(0, GROUPS, group_body, 0)
        pltpu.sync_copy(out_v, sparse_hbm.at[pl.ds(row0 * EXPERTS, CHUNK * EXPERTS)])
        pltpu.sync_copy(idx_v, idx_hbm.at[pl.ds(row0 * TOPK, CHUNK * TOPK)])
        return carry

    lax.fori_loop(0, NCHUNK, chunk_body, 0)


def _route(logits_flat):
    mesh = plsc.VectorSubcoreMesh(
        core_axis_name="c",
        subcore_axis_name="s",
        num_cores=NUM_CORES,
        num_subcores=NUM_SUBCORES,
    )
    fn = pl.kernel(
        _route_body,
        out_type=[
            jax.ShapeDtypeStruct((TOKENS * EXPERTS,), jnp.float32),
            jax.ShapeDtypeStruct((TOKENS * TOPK,), jnp.int32),
        ],
        mesh=mesh,
        scratch_types=[
            pltpu.VMEM((CHUNK * EXPERTS,), jnp.float32),
            pltpu.VMEM((CHUNK * EXPERTS,), jnp.float32),
            pltpu.VMEM((CHUNK * TOPK,), jnp.int32),
        ],
        compiler_params=pltpu.CompilerParams(needs_layout_passes=False),
    )
    return fn(logits_flat)


@jax.jit
def kernel(x, W, b):
    logits = _compute_logits(x, W, b)
    sparse_flat, idx_flat = _route(logits.reshape(-1))
    return (
        sparse_flat.reshape(TOKENS, EXPERTS),
        idx_flat.reshape(TOKENS, TOPK),
        logits,
    )
